# baseline (speedup 1.0000x reference)
import functools

import jax
import jax.numpy as jnp
from jax import lax
from jax.experimental import pallas as pl
from jax.experimental.pallas import tpu as pltpu
from jax.experimental.pallas import tpu_sc as plsc

VOCAB = 1000000
D = 32
B = 16384

NC = 2
NS = 16
L = 16
NW = NC * NS
BPW = B // NW
ICH = 128
NICH = BPW // ICH

P = 1015808
STRIPE = P // NS
ZROWS = STRIPE // 2

RB = 128
NB = 61
TAIL0 = NB * RB * 128
NTAIL = VOCAB - TAIL0


def _sc_counts(idx_hbm, c0_hbm, c1_hbm, idx_v, ones_v, zbuf_v, shared, sem):
    cid = lax.axis_index("c")
    sid = lax.axis_index("s")
    wid = sid * NC + cid

    def zloop(i, _):
        zbuf_v[pl.ds(i * L, L)] = jnp.zeros((L,), jnp.float32)
        return 0

    lax.fori_loop(0, ZROWS // L, zloop, 0, unroll=4)
    for q in range(2):
        pltpu.sync_copy(zbuf_v, shared.at[pl.ds(sid * STRIPE + q * ZROWS, ZROWS)])

    def oloop(i, _):
        ones_v[pl.ds(i * L, L)] = jnp.ones((L,), jnp.float32)
        return 0

    lax.fori_loop(0, ICH // L, oloop, 0, unroll=4)
    pltpu.sync_copy(idx_hbm.at[pl.ds(wid * NICH, NICH)], idx_v)
    plsc.subcore_barrier()

    for j in range(NICH):
        pltpu.sync_copy(ones_v, shared.at[idx_v.at[j]], add=True)
    plsc.subcore_barrier()

    @pl.when(cid == 0)
    def _():
        pltpu.sync_copy(shared.at[pl.ds(sid * STRIPE, STRIPE)],
                        c0_hbm.at[pl.ds(sid * STRIPE, STRIPE)])

    @pl.when(cid == 1)
    def _():
        pltpu.sync_copy(shared.at[pl.ds(sid * STRIPE, STRIPE)],
                        c1_hbm.at[pl.ds(sid * STRIPE, STRIPE)])


_counts_call = functools.partial(
    pl.kernel,
    out_type=(
        jax.ShapeDtypeStruct((P,), jnp.float32),
        jax.ShapeDtypeStruct((P,), jnp.float32),
    ),
    mesh=plsc.VectorSubcoreMesh(core_axis_name="c", subcore_axis_name="s"),
    scratch_types=[
        pltpu.VMEM((NICH, ICH), jnp.int32),
        pltpu.VMEM((ICH,), jnp.float32),
        pltpu.VMEM((ZROWS,), jnp.float32),
        pltpu.VMEM_SHARED((P,), jnp.float32),
        pltpu.SemaphoreType.DMA,
    ],
    compiler_params=pltpu.CompilerParams(use_tc_tiling_on_sc=True),
)(_sc_counts)


def _tc_stream(t_ref, c0_ref, c1_ref, o_ref):
    i = pl.program_id(0)

    @pl.when(i == 0)
    def _():
        o_ref[...] = jnp.zeros_like(o_ref)

    c = c0_ref[...] + c1_ref[...]
    o_ref[...] += jnp.sum(t_ref[...] * c[None, :], axis=1, keepdims=True)


def _tc_combine(ptc_ref, ttail_ref, ctail_ref, w_ref, b_ref, o_ref):
    s = ptc_ref[:, 0]
    ctail = ctail_ref[0, :NTAIL]
    s = s + jnp.sum(ttail_ref[...] * ctail[None, :], axis=1)
    o_ref[...] = (jnp.sum(s * w_ref[:, 0]) * (1.0 / B) + b_ref[0]).reshape(1, 1)


def kernel(inputs, table, W, b):
    idx = inputs.astype(jnp.int32).reshape(NW * NICH, ICH)
    tT = table.T
    c0, c1 = _counts_call(idx)

    ptc = pl.pallas_call(
        _tc_stream,
        grid=(NB,),
        in_specs=[
            pl.BlockSpec((D, RB * 128), lambda j: (0, j)),
            pl.BlockSpec((RB * 128,), lambda j: (j,)),
            pl.BlockSpec((RB * 128,), lambda j: (j,)),
        ],
        out_specs=pl.BlockSpec((D, 1), lambda j: (0, 0)),
        out_shape=jax.ShapeDtypeStruct((D, 1), jnp.float32),
    )(tT, c0, c1)

    ttail = lax.slice(tT, (0, TAIL0), (D, VOCAB))
    ctail = (lax.slice(c0, (TAIL0,), (TAIL0 + 640,))
             + lax.slice(c1, (TAIL0,), (TAIL0 + 640,)))[None, :]

    return pl.pallas_call(
        _tc_combine,
        out_shape=jax.ShapeDtypeStruct((1, 1), jnp.float32),
    )(ptc, ttail, ctail, W, b)


# speedup vs baseline: 1.1377x; 1.1377x over previous
import functools

import jax
import jax.numpy as jnp
from jax import lax
from jax.experimental import pallas as pl
from jax.experimental.pallas import tpu as pltpu
from jax.experimental.pallas import tpu_sc as plsc

VOCAB = 1000000
D = 32
B = 16384

NC = 2
NS = 16
L = 16
NW = NC * NS
BPW = B // NW
ICH = 128
NICH = BPW // ICH

P = 1015808
STRIPE = P // NS
ZROWS = STRIPE // 2

RB = 488
NB = 16
TAIL0 = NB * RB * 128
NTAIL = VOCAB - TAIL0


def _sc_counts(idx_hbm, c0_hbm, c1_hbm, idx_v, ones_v, zbuf_v, shared, sem):
    cid = lax.axis_index("c")
    sid = lax.axis_index("s")
    wid = sid * NC + cid

    def zloop(i, _):
        zbuf_v[pl.ds(i * L, L)] = jnp.zeros((L,), jnp.float32)
        return 0

    lax.fori_loop(0, ZROWS // L, zloop, 0, unroll=4)
    for q in range(2):
        pltpu.sync_copy(zbuf_v, shared.at[pl.ds(sid * STRIPE + q * ZROWS, ZROWS)])

    def oloop(i, _):
        ones_v[pl.ds(i * L, L)] = jnp.ones((L,), jnp.float32)
        return 0

    lax.fori_loop(0, ICH // L, oloop, 0, unroll=4)
    pltpu.sync_copy(idx_hbm.at[pl.ds(wid * NICH, NICH)], idx_v)
    plsc.subcore_barrier()

    for j in range(NICH):
        pltpu.sync_copy(ones_v, shared.at[idx_v.at[j]], add=True)
    plsc.subcore_barrier()

    @pl.when(cid == 0)
    def _():
        pltpu.sync_copy(shared.at[pl.ds(sid * STRIPE, STRIPE)],
                        c0_hbm.at[pl.ds(sid * STRIPE, STRIPE)])

    @pl.when(cid == 1)
    def _():
        pltpu.sync_copy(shared.at[pl.ds(sid * STRIPE, STRIPE)],
                        c1_hbm.at[pl.ds(sid * STRIPE, STRIPE)])


_counts_call = functools.partial(
    pl.kernel,
    out_type=(
        jax.ShapeDtypeStruct((P,), jnp.float32),
        jax.ShapeDtypeStruct((P,), jnp.float32),
    ),
    mesh=plsc.VectorSubcoreMesh(core_axis_name="c", subcore_axis_name="s"),
    scratch_types=[
        pltpu.VMEM((NICH, ICH), jnp.int32),
        pltpu.VMEM((ICH,), jnp.float32),
        pltpu.VMEM((ZROWS,), jnp.float32),
        pltpu.VMEM_SHARED((P,), jnp.float32),
        pltpu.SemaphoreType.DMA,
    ],
    compiler_params=pltpu.CompilerParams(use_tc_tiling_on_sc=True),
)(_sc_counts)


def _tc_stream(t_ref, c0_ref, c1_ref, o_ref):
    i = pl.program_id(0)

    @pl.when(i == 0)
    def _():
        o_ref[...] = jnp.zeros_like(o_ref)

    c = (c0_ref[...] + c1_ref[...]).reshape(RB, 128)
    t = t_ref[...].reshape(D, RB, 128)
    o_ref[...] += jnp.sum(t * c[None, :, :], axis=1)


def _tc_combine(ptc_ref, ttail_ref, ctail_ref, w_ref, b_ref, o_ref):
    s = jnp.sum(ptc_ref[...], axis=1)
    ctail = ctail_ref[0, :NTAIL]
    s = s + jnp.sum(ttail_ref[...] * ctail[None, :], axis=1)
    o_ref[...] = (jnp.sum(s * w_ref[:, 0]) * (1.0 / B) + b_ref[0]).reshape(1, 1)


def kernel(inputs, table, W, b):
    idx = inputs.astype(jnp.int32).reshape(NW * NICH, ICH)
    tT = table.T
    c0, c1 = _counts_call(idx)

    ptc = pl.pallas_call(
        _tc_stream,
        grid=(NB,),
        in_specs=[
            pl.BlockSpec((D, RB * 128), lambda j: (0, j)),
            pl.BlockSpec((RB * 128,), lambda j: (j,)),
            pl.BlockSpec((RB * 128,), lambda j: (j,)),
        ],
        out_specs=pl.BlockSpec((D, 128), lambda j: (0, 0)),
        out_shape=jax.ShapeDtypeStruct((D, 128), jnp.float32),
    )(tT, c0, c1)

    ttail = lax.slice(tT, (0, TAIL0), (D, VOCAB))
    ctail = (lax.slice(c0, (TAIL0,), (TAIL0 + 640,))
             + lax.slice(c1, (TAIL0,), (TAIL0 + 640,)))[None, :]

    return pl.pallas_call(
        _tc_combine,
        out_shape=jax.ShapeDtypeStruct((1, 1), jnp.float32),
    )(ptc, ttail, ctail, W, b)
